# baseline (speedup 1.0000x reference)
import functools

import jax
import jax.numpy as jnp
from jax.experimental import pallas as pl
from jax.experimental.pallas import tpu as pltpu

_TOPK = 64


def _encoder_body(x_ref, w_ref, benc_ref, bdec_ref, out_ref):
    xm = x_ref[...] - bdec_ref[...]
    acts = jax.lax.dot_general(
        xm, w_ref[...],
        dimension_numbers=(((1,), (1,)), ((), ())),
        preferred_element_type=jnp.float32,
    )
    out_ref[...] = jnp.maximum(acts + benc_ref[...], 0.0)


def _threshold_body(p_ref, t_ref, *, k: int):
    rows = p_ref.shape[0]
    ftot = p_ref.shape[1]
    cw = 2048
    nchunks = ftot // cw

    def bit_step(i, t):
        trial = t | (jnp.int32(1) << (jnp.int32(30) - i))

        def chunk_step(j, cnt):
            blk = jax.lax.bitcast_convert_type(
                p_ref[:, pl.ds(j * cw, cw)], jnp.int32)
            return cnt + jnp.sum((blk >= trial).astype(jnp.int32),
                                 axis=1, keepdims=True)

        cnt = jax.lax.fori_loop(0, nchunks, chunk_step,
                                jnp.zeros((rows, 1), jnp.int32))
        return jnp.where(cnt >= k, trial, t)

    t = jax.lax.fori_loop(0, 31, bit_step, jnp.zeros((rows, 1), jnp.int32))
    t_ref[...] = jnp.broadcast_to(t, t_ref.shape)


def _decode_body(p_ref, wd_ref, t_ref, bdec_ref, out_ref):
    f = pl.program_id(0)
    t = t_ref[:, :1]
    pf = p_ref[...]
    pfbits = jax.lax.bitcast_convert_type(pf, jnp.int32)
    e = jnp.where(pfbits >= t, pf, 0.0)
    contrib = jax.lax.dot_general(
        e, wd_ref[...],
        dimension_numbers=(((1,), (1,)), ((), ())),
        preferred_element_type=jnp.float32,
    )

    @pl.when(f == 0)
    def _init():
        out_ref[...] = bdec_ref[...] + contrib

    @pl.when(f > 0)
    def _acc():
        out_ref[...] += contrib


def kernel(x, W_enc, b_enc, W_dec, b_dec):
    B, D = x.shape
    F = W_enc.shape[0]
    benc2 = b_enc.reshape(1, F)
    bdec2 = b_dec.reshape(1, D)

    rb = min(256, B)
    fb = min(2048, F)
    preact = pl.pallas_call(
        _encoder_body,
        grid=(F // fb, B // rb),
        in_specs=[
            pl.BlockSpec((rb, D), lambda f, b: (b, 0)),
            pl.BlockSpec((fb, D), lambda f, b: (f, 0)),
            pl.BlockSpec((1, fb), lambda f, b: (0, f)),
            pl.BlockSpec((1, D), lambda f, b: (0, 0)),
        ],
        out_specs=pl.BlockSpec((rb, fb), lambda f, b: (b, f)),
        out_shape=jax.ShapeDtypeStruct((B, F), jnp.float32),
    )(x, W_enc, benc2, bdec2)

    rt = min(256, B)
    thresh = pl.pallas_call(
        functools.partial(_threshold_body, k=_TOPK),
        grid=(B // rt,),
        in_specs=[pl.BlockSpec((rt, F), lambda b: (b, 0))],
        out_specs=pl.BlockSpec((rt, 128), lambda b: (b, 0)),
        out_shape=jax.ShapeDtypeStruct((B, 128), jnp.int32),
    )(preact)

    return (preact[:, :D] + thresh[:, :1].astype(jnp.float32))
    fb2 = min(512, F)
    x_hat = pl.pallas_call(
        _decode_body,
        grid=(F // fb2,),
        in_specs=[
            pl.BlockSpec((B, fb2), lambda f: (0, f)),
            pl.BlockSpec((D, fb2), lambda f: (0, f)),
            pl.BlockSpec((B, 128), lambda f: (0, 0)),
            pl.BlockSpec((1, D), lambda f: (0, 0)),
        ],
        out_specs=pl.BlockSpec((B, D), lambda f: (0, 0)),
        out_shape=jax.ShapeDtypeStruct((B, D), jnp.float32),
    )(preact, W_dec, thresh, bdec2)
    return x_hat


# speedup vs baseline: 1.0195x; 1.0195x over previous
import functools

import jax
import jax.numpy as jnp
from jax.experimental import pallas as pl
from jax.experimental.pallas import tpu as pltpu

_TOPK = 64


def _encoder_body(x_ref, w_ref, benc_ref, bdec_ref, out_ref):
    w = w_ref[...]
    corr = jax.lax.dot_general(
        bdec_ref[...], w,
        dimension_numbers=(((1,), (1,)), ((), ())),
        preferred_element_type=jnp.float32,
    )
    acts = jax.lax.dot_general(
        x_ref[...], w,
        dimension_numbers=(((1,), (1,)), ((), ())),
        preferred_element_type=jnp.float32,
    )
    out_ref[...] = jnp.maximum(acts - corr + benc_ref[...], 0.0)


def _threshold_body(p_ref, t_ref, *, k: int):
    rows = p_ref.shape[0]
    ftot = p_ref.shape[1]
    cw = 2048
    nchunks = ftot // cw

    def bit_step(i, t):
        trial = t | (jnp.int32(1) << (jnp.int32(30) - i))
        cnt = jnp.zeros((rows, 1), jnp.int32)
        for j in range(nchunks):
            blk = jax.lax.bitcast_convert_type(
                p_ref[:, j * cw:(j + 1) * cw], jnp.int32)
            cnt = cnt + jnp.sum((blk >= trial).astype(jnp.int32),
                                axis=1, keepdims=True)
        return jnp.where(cnt >= k, trial, t)

    t = jax.lax.fori_loop(0, 31, bit_step, jnp.zeros((rows, 1), jnp.int32))
    t_ref[...] = jnp.broadcast_to(t, t_ref.shape)


def _decode_body(p_ref, wd_ref, t_ref, bdec_ref, out_ref):
    f = pl.program_id(0)
    t = t_ref[:, :1]
    pf = p_ref[...]
    pfbits = jax.lax.bitcast_convert_type(pf, jnp.int32)
    e = jnp.where(pfbits >= t, pf, 0.0)
    contrib = jax.lax.dot_general(
        e, wd_ref[...],
        dimension_numbers=(((1,), (1,)), ((), ())),
        preferred_element_type=jnp.float32,
    )

    @pl.when(f == 0)
    def _init():
        out_ref[...] = bdec_ref[...] + contrib

    @pl.when(f > 0)
    def _acc():
        out_ref[...] += contrib


def kernel(x, W_enc, b_enc, W_dec, b_dec):
    B, D = x.shape
    F = W_enc.shape[0]
    benc2 = b_enc.reshape(1, F)
    bdec2 = b_dec.reshape(1, D)

    fb = min(1024, F)
    preact = pl.pallas_call(
        _encoder_body,
        grid=(F // fb,),
        in_specs=[
            pl.BlockSpec((B, D), lambda f: (0, 0)),
            pl.BlockSpec((fb, D), lambda f: (f, 0)),
            pl.BlockSpec((1, fb), lambda f: (0, f)),
            pl.BlockSpec((1, D), lambda f: (0, 0)),
        ],
        out_specs=pl.BlockSpec((B, fb), lambda f: (0, f)),
        out_shape=jax.ShapeDtypeStruct((B, F), jnp.float32),
    )(x, W_enc, benc2, bdec2)

    rt = min(256, B)
    thresh = pl.pallas_call(
        functools.partial(_threshold_body, k=_TOPK),
        grid=(B // rt,),
        in_specs=[pl.BlockSpec((rt, F), lambda b: (b, 0))],
        out_specs=pl.BlockSpec((rt, 128), lambda b: (b, 0)),
        out_shape=jax.ShapeDtypeStruct((B, 128), jnp.int32),
    )(preact)

    fb2 = min(512, F)
    x_hat = pl.pallas_call(
        _decode_body,
        grid=(F // fb2,),
        in_specs=[
            pl.BlockSpec((B, fb2), lambda f: (0, f)),
            pl.BlockSpec((D, fb2), lambda f: (0, f)),
            pl.BlockSpec((B, 128), lambda f: (0, 0)),
            pl.BlockSpec((1, D), lambda f: (0, 0)),
        ],
        out_specs=pl.BlockSpec((B, D), lambda f: (0, 0)),
        out_shape=jax.ShapeDtypeStruct((B, D), jnp.float32),
    )(preact, W_dec, thresh, bdec2)
    return x_hat


# speedup vs baseline: 1.0582x; 1.0380x over previous
import functools

import jax
import jax.numpy as jnp
from jax.experimental import pallas as pl
from jax.experimental.pallas import tpu as pltpu

_TOPK = 64


def _encoder_body(x_ref, w_ref, benc_ref, bdec_ref, out_ref):
    w = w_ref[...]
    corr = jax.lax.dot_general(
        bdec_ref[...], w,
        dimension_numbers=(((1,), (1,)), ((), ())),
        preferred_element_type=jnp.float32,
    )
    acts = jax.lax.dot_general(
        x_ref[...], w,
        dimension_numbers=(((1,), (1,)), ((), ())),
        preferred_element_type=jnp.float32,
    )
    out_ref[...] = jnp.maximum(acts - corr + benc_ref[...], 0.0)


def _threshold_body(p_ref, t_ref, *, k: int):
    rows = p_ref.shape[0]
    ftot = p_ref.shape[1]
    cw = 128
    nchunks = ftot // cw

    def bit_step(i, t):
        trial = t | (jnp.int32(1) << (jnp.int32(30) - i))
        acc = jnp.zeros((rows, cw), jnp.int32)
        for j in range(nchunks):
            blk = jax.lax.bitcast_convert_type(
                p_ref[:, j * cw:(j + 1) * cw], jnp.int32)
            acc = acc + (blk >= trial).astype(jnp.int32)
        cnt = jnp.sum(acc, axis=1, keepdims=True)
        return jnp.where(cnt >= k, trial, t)

    t = jax.lax.fori_loop(0, 31, bit_step, jnp.zeros((rows, 1), jnp.int32))
    t_ref[...] = jnp.broadcast_to(t, t_ref.shape)


def _decode_body(p_ref, wd_ref, t_ref, bdec_ref, out_ref):
    f = pl.program_id(0)
    t = t_ref[:, :1]
    pf = p_ref[...]
    pfbits = jax.lax.bitcast_convert_type(pf, jnp.int32)
    e = jnp.where(pfbits >= t, pf, 0.0)
    contrib = jax.lax.dot_general(
        e, wd_ref[...],
        dimension_numbers=(((1,), (1,)), ((), ())),
        preferred_element_type=jnp.float32,
    )

    @pl.when(f == 0)
    def _init():
        out_ref[...] = bdec_ref[...] + contrib

    @pl.when(f > 0)
    def _acc():
        out_ref[...] += contrib


def kernel(x, W_enc, b_enc, W_dec, b_dec):
    B, D = x.shape
    F = W_enc.shape[0]
    benc2 = b_enc.reshape(1, F)
    bdec2 = b_dec.reshape(1, D)

    fb = min(1024, F)
    preact = pl.pallas_call(
        _encoder_body,
        grid=(F // fb,),
        in_specs=[
            pl.BlockSpec((B, D), lambda f: (0, 0)),
            pl.BlockSpec((fb, D), lambda f: (f, 0)),
            pl.BlockSpec((1, fb), lambda f: (0, f)),
            pl.BlockSpec((1, D), lambda f: (0, 0)),
        ],
        out_specs=pl.BlockSpec((B, fb), lambda f: (0, f)),
        out_shape=jax.ShapeDtypeStruct((B, F), jnp.float32),
    )(x, W_enc, benc2, bdec2)

    rt = min(256, B)
    thresh = pl.pallas_call(
        functools.partial(_threshold_body, k=_TOPK),
        grid=(B // rt,),
        in_specs=[pl.BlockSpec((rt, F), lambda b: (b, 0))],
        out_specs=pl.BlockSpec((rt, 128), lambda b: (b, 0)),
        out_shape=jax.ShapeDtypeStruct((B, 128), jnp.int32),
    )(preact)

    fb2 = min(512, F)
    x_hat = pl.pallas_call(
        _decode_body,
        grid=(F // fb2,),
        in_specs=[
            pl.BlockSpec((B, fb2), lambda f: (0, f)),
            pl.BlockSpec((D, fb2), lambda f: (0, f)),
            pl.BlockSpec((B, 128), lambda f: (0, 0)),
            pl.BlockSpec((1, D), lambda f: (0, 0)),
        ],
        out_specs=pl.BlockSpec((B, D), lambda f: (0, 0)),
        out_shape=jax.ShapeDtypeStruct((B, D), jnp.float32),
    )(preact, W_dec, thresh, bdec2)
    return x_hat
